# baseline (speedup 1.0000x reference)
import functools

import jax
import jax.numpy as jnp
from jax import lax
from jax.experimental import pallas as pl
from jax.experimental.pallas import tpu as pltpu
from jax.experimental.pallas import tpu_sc as plsc

N_TOK = 8192
N_EMB = 8192
D = 256
COMMIT = 0.25

BM = 512
WIN = 4096
LOSS_SCALE = (1.0 + COMMIT) / (N_TOK * D)


def _argmin_body(x_ref, e_ref, x2_ref, e2_ref, idx_ref, loss_ref):
    i = pl.program_id(0)

    x = x_ref[...]
    e = e_ref[...]
    x2 = lax.transpose(jnp.reshape(x2_ref[...], (1, BM)), (1, 0))
    e2 = e2_ref[...]
    m2 = lax.dot_general(x, e, (((1,), (1,)), ((), ())),
                         preferred_element_type=jnp.float32)

    ids = lax.broadcasted_iota(jnp.int32, (1, WIN), 1).astype(jnp.float32)

    da = (x2 - m2[:, :WIN]) + e2[:, :WIN]
    mina = jnp.min(da, axis=1, keepdims=True)
    arga = jnp.min(jnp.where(da == mina, ids, float(WIN)),
                   axis=1, keepdims=True)
    carry = mina.astype(jnp.bfloat16).astype(jnp.float32)

    db = (x2 - m2[:, WIN:]) + e2[:, WIN:]
    minb = jnp.min(db, axis=1, keepdims=True)
    argb = jnp.min(jnp.where(db == minb, ids, float(WIN)),
                   axis=1, keepdims=True)

    better = minb < carry
    fmin = jnp.where(better, minb, carry)
    fmin = fmin.astype(jnp.bfloat16).astype(jnp.float32)
    farg = jnp.where(better, WIN + argb, arga).astype(jnp.int32)

    idx_ref[...] = jnp.reshape(lax.transpose(farg, (1, 0)), (1, 1, BM))
    contrib = jnp.sum(fmin) * LOSS_SCALE

    @pl.when(i == 0)
    def _first():
        loss_ref[0, 0] = contrib

    @pl.when(i > 0)
    def _acc():
        loss_ref[0, 0] += contrib


def _argmin_loss(x, e, x2, e2):
    grid = (N_TOK // BM,)
    return pl.pallas_call(
        _argmin_body,
        grid=grid,
        in_specs=[
            pl.BlockSpec((BM, D), lambda i: (i, 0)),
            pl.BlockSpec((N_EMB, D), lambda i: (0, 0)),
            pl.BlockSpec((1, 1, BM), lambda i: (i, 0, 0)),
            pl.BlockSpec((1, N_EMB), lambda i: (0, 0)),
        ],
        out_specs=[
            pl.BlockSpec((1, 1, BM), lambda i: (i, 0, 0)),
            pl.BlockSpec(memory_space=pltpu.SMEM),
        ],
        out_shape=[
            jax.ShapeDtypeStruct((N_TOK // BM, 1, BM), jnp.int32),
            jax.ShapeDtypeStruct((1, 1), jnp.float32),
        ],
        compiler_params=pltpu.CompilerParams(
            dimension_semantics=("arbitrary",),
        ),
    )(x, e, x2, e2)


_NC = 2
_NS = 16
_NW = _NC * _NS
_BPW = N_TOK // _NW
_CHUNK = 128
_NCH = _BPW // _CHUNK


def _sc_gather(table, idx3):
    mesh = plsc.VectorSubcoreMesh(core_axis_name="c", subcore_axis_name="s")

    @functools.partial(
        pl.kernel,
        mesh=mesh,
        out_type=jax.ShapeDtypeStruct((N_TOK, D), jnp.float32),
        scratch_types=[
            pltpu.VMEM((_NCH, _CHUNK), jnp.int32),
            pltpu.VMEM((_BPW, D), jnp.float32),
            pltpu.SemaphoreType.DMA,
        ],
    )
    def k(table_hbm, idx_hbm, out_hbm, idx_v, rows_v, sem):
        wid = lax.axis_index("s") * _NC + lax.axis_index("c")
        base = wid * _BPW
        pltpu.sync_copy(idx_hbm.at[wid], idx_v)
        copies = [
            pltpu.async_copy(
                table_hbm.at[idx_v.at[c]],
                rows_v.at[pl.ds(c * _CHUNK, _CHUNK)],
                sem,
            )
            for c in range(_NCH)
        ]
        for cp in copies:
            cp.wait()
        pltpu.sync_copy(rows_v, out_hbm.at[pl.ds(base, _BPW)])

    return k(table, idx3)


def kernel(inputs, embedding):
    x_bf = (2.0 * jnp.reshape(inputs, (N_TOK, D))).astype(jnp.bfloat16)
    e_bf = embedding.astype(jnp.bfloat16)
    x2 = jnp.reshape(jnp.sum(inputs ** 2, axis=2), (N_TOK // BM, 1, BM))
    e2 = jnp.reshape(jnp.sum(embedding ** 2, axis=1), (1, N_EMB))
    idx2d, loss11 = _argmin_loss(x_bf, e_bf, x2, e2)
    idx = jnp.reshape(idx2d, (N_TOK,))
    idx3 = jnp.reshape(idx, (_NW, _NCH, _CHUNK))
    table = lax.optimization_barrier(
        embedding.astype(jnp.bfloat16)).astype(jnp.float32)
    quant = _sc_gather(table, idx3)
    quantized = jnp.reshape(quant, inputs.shape)
    vq_loss = loss11[0, 0]
    return quantized, vq_loss, idx


# speedup vs baseline: 1.0357x; 1.0357x over previous
import functools

import jax
import jax.numpy as jnp
from jax import lax
from jax.experimental import pallas as pl
from jax.experimental.pallas import tpu as pltpu
from jax.experimental.pallas import tpu_sc as plsc

N_TOK = 8192
N_EMB = 8192
D = 256
COMMIT = 0.25

BM = 1024
BN = 4096
LOSS_SCALE = (1.0 + COMMIT) / (N_TOK * D)


def _argmin_body(x_ref, e_ref, x2_ref, e2_ref, idx_ref, loss_ref, minv, argv):
    j = pl.program_id(1)
    nj = pl.num_programs(1)

    x = x_ref[...]
    e = e_ref[...]
    x2 = lax.transpose(jnp.reshape(x2_ref[...], (1, BM)), (1, 0))
    e2 = e2_ref[...]
    m2 = lax.dot_general(x, e, (((1,), (1,)), ((), ())),
                         preferred_element_type=jnp.float32)
    d = (x2 - m2) + e2

    bmin = jnp.min(d, axis=1, keepdims=True)
    ids = lax.broadcasted_iota(jnp.int32, (1, BN), 1).astype(jnp.float32)
    barg = jnp.min(jnp.where(d == bmin, ids, float(BN)),
                   axis=1, keepdims=True)
    gidx = j * BN + barg.astype(jnp.int32)

    @pl.when(j == 0)
    def _init():
        minv[...] = bmin
        argv[...] = gidx

    @pl.when(j > 0)
    def _update():
        better = bmin < minv[...]
        minv[...] = jnp.where(better, bmin, minv[...])
        argv[...] = jnp.where(better, gidx, argv[...])

    minv[...] = minv[...].astype(jnp.bfloat16).astype(jnp.float32)

    @pl.when(j == nj - 1)
    def _finalize():
        idx_ref[...] = jnp.reshape(
            lax.transpose(argv[...], (1, 0)), (1, 1, BM))
        contrib = jnp.sum(minv[...]) * LOSS_SCALE

        @pl.when(pl.program_id(0) == 0)
        def _first():
            loss_ref[0, 0] = contrib

        @pl.when(pl.program_id(0) > 0)
        def _acc():
            loss_ref[0, 0] += contrib


def _argmin_loss(x, e, x2, e2):
    grid = (N_TOK // BM, N_EMB // BN)
    return pl.pallas_call(
        _argmin_body,
        grid=grid,
        in_specs=[
            pl.BlockSpec((BM, D), lambda i, j: (i, 0)),
            pl.BlockSpec((BN, D), lambda i, j: (j, 0)),
            pl.BlockSpec((1, 1, BM), lambda i, j: (i, 0, 0)),
            pl.BlockSpec((1, BN), lambda i, j: (0, j)),
        ],
        out_specs=[
            pl.BlockSpec((1, 1, BM), lambda i, j: (i, 0, 0)),
            pl.BlockSpec(memory_space=pltpu.SMEM),
        ],
        out_shape=[
            jax.ShapeDtypeStruct((N_TOK // BM, 1, BM), jnp.int32),
            jax.ShapeDtypeStruct((1, 1), jnp.float32),
        ],
        scratch_shapes=[
            pltpu.VMEM((BM, 1), jnp.float32),
            pltpu.VMEM((BM, 1), jnp.int32),
        ],
        compiler_params=pltpu.CompilerParams(
            dimension_semantics=("arbitrary", "arbitrary"),
        ),
    )(x, e, x2, e2)


_NC = 2
_NS = 16
_NW = _NC * _NS
_BPW = N_TOK // _NW
_CHUNK = 128
_NCH = _BPW // _CHUNK


def _sc_gather(table, idx3):
    mesh = plsc.VectorSubcoreMesh(core_axis_name="c", subcore_axis_name="s")

    @functools.partial(
        pl.kernel,
        mesh=mesh,
        out_type=jax.ShapeDtypeStruct((N_TOK, D), jnp.float32),
        scratch_types=[
            pltpu.VMEM((_NCH, _CHUNK), jnp.int32),
            pltpu.VMEM((_BPW, D), jnp.float32),
            pltpu.SemaphoreType.DMA,
        ],
    )
    def k(table_hbm, idx_hbm, out_hbm, idx_v, rows_v, sem):
        wid = lax.axis_index("s") * _NC + lax.axis_index("c")
        base = wid * _BPW
        pltpu.sync_copy(idx_hbm.at[wid], idx_v)
        copies = [
            pltpu.async_copy(
                table_hbm.at[idx_v.at[c]],
                rows_v.at[pl.ds(c * _CHUNK, _CHUNK)],
                sem,
            )
            for c in range(_NCH)
        ]
        for cp in copies:
            cp.wait()
        pltpu.sync_copy(rows_v, out_hbm.at[pl.ds(base, _BPW)])

    return k(table, idx3)


def kernel(inputs, embedding):
    x_bf = (2.0 * jnp.reshape(inputs, (N_TOK, D))).astype(jnp.bfloat16)
    e_bf = embedding.astype(jnp.bfloat16)
    x2 = jnp.reshape(jnp.sum(inputs ** 2, axis=2), (N_TOK // BM, 1, BM))
    e2 = jnp.reshape(jnp.sum(embedding ** 2, axis=1), (1, N_EMB))
    idx2d, loss11 = _argmin_loss(x_bf, e_bf, x2, e2)
    idx = jnp.reshape(idx2d, (N_TOK,))
    idx3 = jnp.reshape(idx, (_NW, _NCH, _CHUNK))
    table = lax.optimization_barrier(
        embedding.astype(jnp.bfloat16)).astype(jnp.float32)
    quant = _sc_gather(table, idx3)
    quantized = jnp.reshape(quant, inputs.shape)
    vq_loss = loss11[0, 0]
    return quantized, vq_loss, idx
